# baseline (speedup 1.0000x reference)
import jax
import jax.numpy as jnp
from jax import lax
from jax.experimental import pallas as pl
from jax.experimental.pallas import tpu as pltpu
from jax.experimental.pallas import tpu_sc as plsc

N = 10000
E = 160000
DIN = 256
HID = 512
OUT = 256
K = 512

NT = 16
NC = 2
CH = 128
NPAD = 10240
EPAD = NC * NT * 40 * CH
EPT_S = EPAD // (NC * NT)
EPT_A = EPAD // NT
PAIR = K * K
PAIRPAD = PAIR + NT * 128
PZC = PAIRPAD // NT
KC = K + 128
DEGC = NPAD // NT
ROWS_PT = NPAD // NT
BM = 2000
GRID = N // BM

_mesh = plsc.VectorSubcoreMesh(core_axis_name="c", subcore_axis_name="s")


def _stats_body(src_hbm, dst_hbm, clu_hbm, z1_hbm,
                deg_out, pair_out, cnt_out,
                spm_deg, spm_pair, spm_cnt,
                sa_v, sb_v, da_v, db_v, cua_v, cub_v, cva_v, cvb_v, ones_v,
                sem_sa, sem_sb, sem_da, sem_db,
                sem_ga, sem_gb, sem_ha, sem_hb):
    c = lax.axis_index("c")
    s = lax.axis_index("s")
    for j in range(CH // 16):
        ones_v[pl.ds(j * 16, 16)] = jnp.full((16,), 1.0, jnp.float32)

    pltpu.sync_copy(z1_hbm.at[pl.ds(0, DEGC)],
                    spm_deg.at[pl.ds(s * DEGC, DEGC)])
    pltpu.sync_copy(z1_hbm.at[pl.ds(0, PZC)],
                    spm_pair.at[pl.ds(s * PZC, PZC)])

    @pl.when(s == 0)
    def _():
        pltpu.sync_copy(z1_hbm.at[pl.ds(0, KC)], spm_cnt)

    plsc.subcore_barrier()

    ebase = (c * NT + s) * EPT_S

    def keys(cu_v, cv_v):
        for j in range(CH // 16):
            sl = pl.ds(j * 16, 16)
            cu = cu_v[sl]
            cv = cv_v[sl]
            cu_v[sl] = jnp.where(cu == cv, PAIR, cv * K + cu)

    def body(i, carry):
        base = ebase + i * (2 * CH)
        la = pltpu.async_copy(src_hbm.at[pl.ds(base, CH)], sa_v, sem_sa)
        lda = pltpu.async_copy(dst_hbm.at[pl.ds(base, CH)], da_v, sem_da)
        lb = pltpu.async_copy(src_hbm.at[pl.ds(base + CH, CH)], sb_v, sem_sb)
        ldb = pltpu.async_copy(dst_hbm.at[pl.ds(base + CH, CH)], db_v, sem_db)
        la.wait()
        ga = pltpu.async_copy(clu_hbm.at[sa_v], cua_v, sem_ga)
        lda.wait()
        ha = pltpu.async_copy(clu_hbm.at[da_v], cva_v, sem_ha)
        lb.wait()
        gb = pltpu.async_copy(clu_hbm.at[sb_v], cub_v, sem_gb)
        ldb.wait()
        hb = pltpu.async_copy(clu_hbm.at[db_v], cvb_v, sem_hb)
        pltpu.sync_copy(ones_v, spm_deg.at[da_v], add=True)
        pltpu.sync_copy(ones_v, spm_deg.at[db_v], add=True)
        ga.wait()
        ha.wait()
        keys(cua_v, cva_v)
        pltpu.sync_copy(ones_v, spm_pair.at[cua_v], add=True)
        gb.wait()
        hb.wait()
        keys(cub_v, cvb_v)
        pltpu.sync_copy(ones_v, spm_pair.at[cub_v], add=True)
        return carry

    lax.fori_loop(0, EPT_S // (2 * CH), body, 0)

    @pl.when(c == 1)
    def _cnt():
        def cbody(i, carry):
            base = s * DEGC + i * CH
            pltpu.sync_copy(clu_hbm.at[pl.ds(base, CH)], sa_v)
            pltpu.sync_copy(ones_v, spm_cnt.at[sa_v], add=True)
            return carry
        lax.fori_loop(0, DEGC // CH, cbody, 0)

    plsc.subcore_barrier()

    pltpu.sync_copy(spm_deg.at[pl.ds(s * DEGC, DEGC)],
                    deg_out.at[pl.ds(c * NPAD + s * DEGC, DEGC)])
    pltpu.sync_copy(spm_pair.at[pl.ds(s * PZC, PZC)],
                    pair_out.at[pl.ds(c * PAIRPAD + s * PZC, PZC)])

    @pl.when(s == 0)
    def _():
        pltpu.sync_copy(spm_cnt, cnt_out.at[pl.ds(c * KC, KC)])


_stats = pl.kernel(
    _stats_body,
    out_type=(
        jax.ShapeDtypeStruct((NC * NPAD,), jnp.float32),
        jax.ShapeDtypeStruct((NC * PAIRPAD,), jnp.float32),
        jax.ShapeDtypeStruct((NC * KC,), jnp.float32),
    ),
    mesh=_mesh,
    scratch_types=[
        pltpu.VMEM_SHARED((NPAD,), jnp.float32),
        pltpu.VMEM_SHARED((PAIRPAD,), jnp.float32),
        pltpu.VMEM_SHARED((KC,), jnp.float32),
        pltpu.VMEM((CH,), jnp.int32),
        pltpu.VMEM((CH,), jnp.int32),
        pltpu.VMEM((CH,), jnp.int32),
        pltpu.VMEM((CH,), jnp.int32),
        pltpu.VMEM((CH,), jnp.int32),
        pltpu.VMEM((CH,), jnp.int32),
        pltpu.VMEM((CH,), jnp.int32),
        pltpu.VMEM((CH,), jnp.int32),
        pltpu.VMEM((CH,), jnp.float32),
        pltpu.SemaphoreType.DMA,
        pltpu.SemaphoreType.DMA,
        pltpu.SemaphoreType.DMA,
        pltpu.SemaphoreType.DMA,
        pltpu.SemaphoreType.DMA,
        pltpu.SemaphoreType.DMA,
        pltpu.SemaphoreType.DMA,
        pltpu.SemaphoreType.DMA,
    ],
)


DEPTH = 4
CHA = 80


def _agg_body(src_hbm, dst_hbm, x2_hbm, z2_hbm, agg_out, spm_agg, *sc):
    sidx = sc[0:DEPTH]
    didx = sc[DEPTH:2 * DEPTH]
    rows = sc[2 * DEPTH:3 * DEPTH]
    sem_ls, sem_ld, sem_g, sem_sc = sc[3 * DEPTH:3 * DEPTH + 4]
    c = lax.axis_index("c")
    s = lax.axis_index("s")
    pltpu.sync_copy(z2_hbm, spm_agg.at[pl.ds(s * ROWS_PT, ROWS_PT)])
    plsc.subcore_barrier()
    off = c * NPAD
    ebase = s * EPT_A

    def addoff(idx_v):
        for j in range(CHA // 16):
            sl = pl.ds(j * 16, 16)
            idx_v[sl] = idx_v[sl] + off

    def body(i, carry):
        base = ebase + i * (DEPTH * CHA)
        ls = []
        ld = []
        for g in range(DEPTH):
            b = base + g * CHA
            ls.append(pltpu.async_copy(src_hbm.at[pl.ds(b, CHA)],
                                       sidx[g], sem_ls))
            ld.append(pltpu.async_copy(dst_hbm.at[pl.ds(b, CHA)],
                                       didx[g], sem_ld))
        gd = []
        for g in range(DEPTH):
            ls[g].wait()
            addoff(sidx[g])
            gd.append(pltpu.async_copy(x2_hbm.at[sidx[g]], rows[g], sem_g))
        scs = []
        for g in range(DEPTH):
            gd[g].wait()
            ld[g].wait()
            scs.append(pltpu.async_copy(rows[g], spm_agg.at[didx[g]],
                                        sem_sc, add=True))
        for g in range(DEPTH):
            scs[g].wait()
        return carry

    lax.fori_loop(0, EPT_A // (DEPTH * CHA), body, 0)
    plsc.subcore_barrier()
    pltpu.sync_copy(spm_agg.at[pl.ds(s * ROWS_PT, ROWS_PT)],
                    agg_out.at[pl.ds(c * NPAD + s * ROWS_PT, ROWS_PT)])


_agg = pl.kernel(
    _agg_body,
    out_type=jax.ShapeDtypeStruct((NC * NPAD, 128), jnp.float32),
    mesh=_mesh,
    scratch_types=(
        [pltpu.VMEM_SHARED((NPAD, 128), jnp.float32)]
        + [pltpu.VMEM((CHA,), jnp.int32) for _ in range(2 * DEPTH)]
        + [pltpu.VMEM((CHA, 128), jnp.float32) for _ in range(DEPTH)]
        + [pltpu.SemaphoreType.DMA for _ in range(4)]
    ),
)


def _prep_body(x_ref, d0_ref, d1_ref, x2_ref, dinv_ref):
    dinv = lax.rsqrt(d0_ref[...] + d1_ref[...] + 1.0)
    xs = x_ref[...] * dinv
    x2_ref[0] = xs[:, :128]
    x2_ref[1] = xs[:, 128:]
    dinv_ref[...] = dinv


_prep = pl.pallas_call(
    _prep_body,
    grid=(GRID,),
    in_specs=[
        pl.BlockSpec((BM, DIN), lambda i: (i, 0)),
        pl.BlockSpec((BM, 1), lambda i: (i, 0)),
        pl.BlockSpec((BM, 1), lambda i: (i, 0)),
    ],
    out_specs=[
        pl.BlockSpec((2, BM, 128), lambda i: (0, i, 0)),
        pl.BlockSpec((BM, 1), lambda i: (i, 0)),
    ],
    out_shape=[
        jax.ShapeDtypeStruct((2, NPAD, 128), jnp.float32),
        jax.ShapeDtypeStruct((N, 1), jnp.float32),
    ],
)


def _main_body(x_ref, agg_ref, dinv_ref, w1_ref, b1_ref, ws_ref, bs_ref,
               cid_ref, skip_ref, sums_ref):
    i = pl.program_id(0)
    dinv = dinv_ref[...]
    agg = jnp.concatenate([agg_ref[0], agg_ref[1]], axis=1)
    pre = dinv * (agg + dinv * x_ref[...])
    h = jnp.dot(pre, w1_ref[...], preferred_element_type=jnp.float32)
    x1 = jnp.maximum(h + b1_ref[...], 0.0)
    skip_ref[...] = jnp.dot(x1, ws_ref[...],
                            preferred_element_type=jnp.float32) + bs_ref[...]
    cid = cid_ref[0, 0, :]
    oh = (lax.broadcasted_iota(jnp.int32, (K, BM), 0)
          == cid[None, :]).astype(jnp.float32)
    contrib = jnp.dot(oh, x1, preferred_element_type=jnp.float32)

    @pl.when(i == 0)
    def _():
        sums_ref[...] = jnp.zeros_like(sums_ref)

    sums_ref[...] += contrib


_main = pl.pallas_call(
    _main_body,
    grid=(GRID,),
    in_specs=[
        pl.BlockSpec((BM, DIN), lambda i: (i, 0)),
        pl.BlockSpec((2, BM, 128), lambda i: (0, i, 0)),
        pl.BlockSpec((BM, 1), lambda i: (i, 0)),
        pl.BlockSpec((DIN, HID), lambda i: (0, 0)),
        pl.BlockSpec((1, HID), lambda i: (0, 0)),
        pl.BlockSpec((HID, OUT), lambda i: (0, 0)),
        pl.BlockSpec((1, OUT), lambda i: (0, 0)),
        pl.BlockSpec((1, 1, BM), lambda i: (i, 0, 0)),
    ],
    out_specs=[
        pl.BlockSpec((BM, OUT), lambda i: (i, 0)),
        pl.BlockSpec((K, HID), lambda i: (0, 0)),
    ],
    out_shape=[
        jax.ShapeDtypeStruct((N, OUT), jnp.float32),
        jax.ShapeDtypeStruct((K, HID), jnp.float32),
    ],
)


def _out_body(sums_ref, c0_ref, c1_ref, pair_ref, w2_ref, b2_ref,
              cid_ref, skip_ref, out_ref, z_s):
    i = pl.program_id(0)

    @pl.when(i == 0)
    def _():
        cnt = jnp.maximum(c0_ref[...] + c1_ref[...], 1.0)
        xp = sums_ref[...] / cnt
        a = jnp.minimum(pair_ref[0] + pair_ref[1], 1.0)
        degp = jnp.sum(a, axis=1, keepdims=True) + 1.0
        dinvp = lax.rsqrt(degp)
        xw = jnp.dot(xp, w2_ref[...], preferred_element_type=jnp.float32)
        t = dinvp * xw
        z_s[...] = dinvp * (jnp.dot(a, t, preferred_element_type=jnp.float32)
                            + t) + b2_ref[...]

    cid = cid_ref[0, 0, :]
    oh = (cid[:, None] == lax.broadcasted_iota(jnp.int32, (BM, K), 1)
          ).astype(jnp.float32)
    up = jnp.dot(oh, z_s[...], preferred_element_type=jnp.float32)
    out_ref[...] = up + skip_ref[...]


_out = pl.pallas_call(
    _out_body,
    grid=(GRID,),
    in_specs=[
        pl.BlockSpec((K, HID), lambda i: (0, 0)),
        pl.BlockSpec((K, 1), lambda i: (0, 0)),
        pl.BlockSpec((K, 1), lambda i: (0, 0)),
        pl.BlockSpec((2, K, K), lambda i: (0, 0, 0)),
        pl.BlockSpec((HID, OUT), lambda i: (0, 0)),
        pl.BlockSpec((1, OUT), lambda i: (0, 0)),
        pl.BlockSpec((1, 1, BM), lambda i: (i, 0, 0)),
        pl.BlockSpec((BM, OUT), lambda i: (i, 0)),
    ],
    out_specs=pl.BlockSpec((BM, OUT), lambda i: (i, 0)),
    out_shape=jax.ShapeDtypeStruct((N, OUT), jnp.float32),
    scratch_shapes=[pltpu.VMEM((K, OUT), jnp.float32)],
)


def kernel(x, edge_index, cluster_id, W1, b1, W2, b2, Ws, bs):
    src = jnp.concatenate(
        [edge_index[0], jnp.zeros((EPAD - E,), jnp.int32)])
    dst = jnp.concatenate(
        [edge_index[1], jnp.full((EPAD - E,), N, jnp.int32)])
    clu_pad = jnp.concatenate(
        [cluster_id, jnp.full((NPAD - N,), K, jnp.int32)])
    z1 = jnp.zeros((PZC,), jnp.float32)
    deg2, pair2, cnt2 = _stats(src, dst, clu_pad, z1)

    x2, dinv = _prep(x,
                     deg2[:N].reshape(N, 1),
                     deg2[NPAD:NPAD + N].reshape(N, 1))

    z2 = jnp.zeros((ROWS_PT, 128), jnp.float32)
    agg_flat = _agg(src, dst, x2.reshape(NC * NPAD, 128), z2)
    agg2 = agg_flat.reshape(NC, NPAD, 128)

    cid3 = cluster_id.reshape(GRID, 1, BM)
    skip, sums = _main(x, agg2, dinv, W1, b1.reshape(1, HID),
                       Ws, bs.reshape(1, OUT), cid3)

    pair3 = pair2.reshape(NC, PAIRPAD // K, K)
    logits = _out(sums,
                  cnt2[:K].reshape(K, 1),
                  cnt2[KC:KC + K].reshape(K, 1),
                  pair3,
                  W2, b2.reshape(1, OUT),
                  cid3, skip)
    return (logits, 0.0)


# speedup vs baseline: 1.0078x; 1.0078x over previous
import jax
import jax.numpy as jnp
from jax import lax
from jax.experimental import pallas as pl
from jax.experimental.pallas import tpu as pltpu
from jax.experimental.pallas import tpu_sc as plsc

N = 10000
E = 160000
DIN = 256
HID = 512
OUT = 256
K = 512

NT = 16
NC = 2
CH = 128
NPAD = 10240
EPAD = NC * NT * 40 * CH
EPT_S = EPAD // (NC * NT)
EPT_A = EPAD // NT
PAIR = K * K
PAIRPAD = PAIR + NT * 128
PZC = PAIRPAD // NT
KC = K + 128
DEGC = NPAD // NT
ROWS_PT = NPAD // NT
BM = 2000
GRID = N // BM

_mesh = plsc.VectorSubcoreMesh(core_axis_name="c", subcore_axis_name="s")


SDEPTH = 4


def _stats_body(src_hbm, dst_hbm, clu_hbm, z1_hbm,
                deg_out, pair_out, cnt_out,
                spm_deg, spm_pair, spm_cnt, *sc):
    sidx = sc[0:SDEPTH]
    didx = sc[SDEPTH:2 * SDEPTH]
    cuv = sc[2 * SDEPTH:3 * SDEPTH]
    cvv = sc[3 * SDEPTH:4 * SDEPTH]
    ones_v = sc[4 * SDEPTH]
    sem_ls, sem_ld, sem_gu, sem_gv = sc[4 * SDEPTH + 1:4 * SDEPTH + 5]
    c = lax.axis_index("c")
    s = lax.axis_index("s")
    for j in range(CH // 16):
        ones_v[pl.ds(j * 16, 16)] = jnp.full((16,), 1.0, jnp.float32)

    pltpu.sync_copy(z1_hbm.at[pl.ds(0, DEGC)],
                    spm_deg.at[pl.ds(s * DEGC, DEGC)])
    pltpu.sync_copy(z1_hbm.at[pl.ds(0, PZC)],
                    spm_pair.at[pl.ds(s * PZC, PZC)])

    @pl.when(s == 0)
    def _():
        pltpu.sync_copy(z1_hbm.at[pl.ds(0, KC)], spm_cnt)

    plsc.subcore_barrier()

    ebase = (c * NT + s) * EPT_S

    def keys(cu_v, cv_v):
        for j in range(CH // 16):
            sl = pl.ds(j * 16, 16)
            cu = cu_v[sl]
            cv = cv_v[sl]
            cu_v[sl] = jnp.where(cu == cv, PAIR, cv * K + cu)

    def body(i, carry):
        base = ebase + i * (SDEPTH * CH)
        ls = []
        ld = []
        for g in range(SDEPTH):
            b = base + g * CH
            ls.append(pltpu.async_copy(src_hbm.at[pl.ds(b, CH)],
                                       sidx[g], sem_ls))
            ld.append(pltpu.async_copy(dst_hbm.at[pl.ds(b, CH)],
                                       didx[g], sem_ld))
        gu = []
        gv = []
        for g in range(SDEPTH):
            ls[g].wait()
            gu.append(pltpu.async_copy(clu_hbm.at[sidx[g]], cuv[g], sem_gu))
            ld[g].wait()
            gv.append(pltpu.async_copy(clu_hbm.at[didx[g]], cvv[g], sem_gv))
        for g in range(SDEPTH):
            pltpu.sync_copy(ones_v, spm_deg.at[didx[g]], add=True)
        for g in range(SDEPTH):
            gu[g].wait()
            gv[g].wait()
            keys(cuv[g], cvv[g])
            pltpu.sync_copy(ones_v, spm_pair.at[cuv[g]], add=True)
        return carry

    lax.fori_loop(0, EPT_S // (SDEPTH * CH), body, 0)

    @pl.when(c == 1)
    def _cnt():
        def cbody(i, carry):
            base = s * DEGC + i * CH
            pltpu.sync_copy(clu_hbm.at[pl.ds(base, CH)], sidx[0])
            pltpu.sync_copy(ones_v, spm_cnt.at[sidx[0]], add=True)
            return carry
        lax.fori_loop(0, DEGC // CH, cbody, 0)

    plsc.subcore_barrier()

    pltpu.sync_copy(spm_deg.at[pl.ds(s * DEGC, DEGC)],
                    deg_out.at[pl.ds(c * NPAD + s * DEGC, DEGC)])
    pltpu.sync_copy(spm_pair.at[pl.ds(s * PZC, PZC)],
                    pair_out.at[pl.ds(c * PAIRPAD + s * PZC, PZC)])

    @pl.when(s == 0)
    def _():
        pltpu.sync_copy(spm_cnt, cnt_out.at[pl.ds(c * KC, KC)])


_stats = pl.kernel(
    _stats_body,
    out_type=(
        jax.ShapeDtypeStruct((NC * NPAD,), jnp.float32),
        jax.ShapeDtypeStruct((NC * PAIRPAD,), jnp.float32),
        jax.ShapeDtypeStruct((NC * KC,), jnp.float32),
    ),
    mesh=_mesh,
    scratch_types=(
        [pltpu.VMEM_SHARED((NPAD,), jnp.float32),
         pltpu.VMEM_SHARED((PAIRPAD,), jnp.float32),
         pltpu.VMEM_SHARED((KC,), jnp.float32)]
        + [pltpu.VMEM((CH,), jnp.int32) for _ in range(4 * SDEPTH)]
        + [pltpu.VMEM((CH,), jnp.float32)]
        + [pltpu.SemaphoreType.DMA for _ in range(4)]
    ),
)


DEPTH = 4
CHA = 80


def _agg_body(src_hbm, dst_hbm, x2_hbm, z2_hbm, agg_out, spm_agg, *sc):
    sidx = sc[0:DEPTH]
    didx = sc[DEPTH:2 * DEPTH]
    rows = sc[2 * DEPTH:3 * DEPTH]
    sem_ls, sem_ld, sem_g, sem_sc = sc[3 * DEPTH:3 * DEPTH + 4]
    c = lax.axis_index("c")
    s = lax.axis_index("s")
    pltpu.sync_copy(z2_hbm, spm_agg.at[pl.ds(s * ROWS_PT, ROWS_PT)])
    plsc.subcore_barrier()
    off = c * NPAD
    ebase = s * EPT_A

    def addoff(idx_v):
        for j in range(CHA // 16):
            sl = pl.ds(j * 16, 16)
            idx_v[sl] = idx_v[sl] + off

    def body(i, carry):
        base = ebase + i * (DEPTH * CHA)
        ls = []
        ld = []
        for g in range(DEPTH):
            b = base + g * CHA
            ls.append(pltpu.async_copy(src_hbm.at[pl.ds(b, CHA)],
                                       sidx[g], sem_ls))
            ld.append(pltpu.async_copy(dst_hbm.at[pl.ds(b, CHA)],
                                       didx[g], sem_ld))
        gd = []
        for g in range(DEPTH):
            ls[g].wait()
            addoff(sidx[g])
            gd.append(pltpu.async_copy(x2_hbm.at[sidx[g]], rows[g], sem_g))
        scs = []
        for g in range(DEPTH):
            gd[g].wait()
            ld[g].wait()
            scs.append(pltpu.async_copy(rows[g], spm_agg.at[didx[g]],
                                        sem_sc, add=True))
        for g in range(DEPTH):
            scs[g].wait()
        return carry

    lax.fori_loop(0, EPT_A // (DEPTH * CHA), body, 0)
    plsc.subcore_barrier()
    pltpu.sync_copy(spm_agg.at[pl.ds(s * ROWS_PT, ROWS_PT)],
                    agg_out.at[pl.ds(c * NPAD + s * ROWS_PT, ROWS_PT)])


_agg = pl.kernel(
    _agg_body,
    out_type=jax.ShapeDtypeStruct((NC * NPAD, 128), jnp.float32),
    mesh=_mesh,
    scratch_types=(
        [pltpu.VMEM_SHARED((NPAD, 128), jnp.float32)]
        + [pltpu.VMEM((CHA,), jnp.int32) for _ in range(2 * DEPTH)]
        + [pltpu.VMEM((CHA, 128), jnp.float32) for _ in range(DEPTH)]
        + [pltpu.SemaphoreType.DMA for _ in range(4)]
    ),
)


def _prep_body(x_ref, d0_ref, d1_ref, x2_ref, dinv_ref):
    dinv = lax.rsqrt(d0_ref[...] + d1_ref[...] + 1.0)
    xs = x_ref[...] * dinv
    x2_ref[0] = xs[:, :128]
    x2_ref[1] = xs[:, 128:]
    dinv_ref[...] = dinv


_prep = pl.pallas_call(
    _prep_body,
    grid=(GRID,),
    in_specs=[
        pl.BlockSpec((BM, DIN), lambda i: (i, 0)),
        pl.BlockSpec((BM, 1), lambda i: (i, 0)),
        pl.BlockSpec((BM, 1), lambda i: (i, 0)),
    ],
    out_specs=[
        pl.BlockSpec((2, BM, 128), lambda i: (0, i, 0)),
        pl.BlockSpec((BM, 1), lambda i: (i, 0)),
    ],
    out_shape=[
        jax.ShapeDtypeStruct((2, NPAD, 128), jnp.float32),
        jax.ShapeDtypeStruct((N, 1), jnp.float32),
    ],
)


def _main_body(x_ref, agg_ref, dinv_ref, w1_ref, b1_ref, ws_ref, bs_ref,
               cid_ref, skip_ref, sums_ref):
    i = pl.program_id(0)
    dinv = dinv_ref[...]
    agg = jnp.concatenate([agg_ref[0], agg_ref[1]], axis=1)
    pre = dinv * (agg + dinv * x_ref[...])
    h = jnp.dot(pre, w1_ref[...], preferred_element_type=jnp.float32)
    x1 = jnp.maximum(h + b1_ref[...], 0.0)
    skip_ref[...] = jnp.dot(x1, ws_ref[...],
                            preferred_element_type=jnp.float32) + bs_ref[...]
    cid = cid_ref[0, 0, :]
    oh = (lax.broadcasted_iota(jnp.int32, (K, BM), 0)
          == cid[None, :]).astype(jnp.float32)
    contrib = jnp.dot(oh, x1, preferred_element_type=jnp.float32)

    @pl.when(i == 0)
    def _():
        sums_ref[...] = jnp.zeros_like(sums_ref)

    sums_ref[...] += contrib


_main = pl.pallas_call(
    _main_body,
    grid=(GRID,),
    in_specs=[
        pl.BlockSpec((BM, DIN), lambda i: (i, 0)),
        pl.BlockSpec((2, BM, 128), lambda i: (0, i, 0)),
        pl.BlockSpec((BM, 1), lambda i: (i, 0)),
        pl.BlockSpec((DIN, HID), lambda i: (0, 0)),
        pl.BlockSpec((1, HID), lambda i: (0, 0)),
        pl.BlockSpec((HID, OUT), lambda i: (0, 0)),
        pl.BlockSpec((1, OUT), lambda i: (0, 0)),
        pl.BlockSpec((1, 1, BM), lambda i: (i, 0, 0)),
    ],
    out_specs=[
        pl.BlockSpec((BM, OUT), lambda i: (i, 0)),
        pl.BlockSpec((K, HID), lambda i: (0, 0)),
    ],
    out_shape=[
        jax.ShapeDtypeStruct((N, OUT), jnp.float32),
        jax.ShapeDtypeStruct((K, HID), jnp.float32),
    ],
)


def _out_body(sums_ref, c0_ref, c1_ref, pair_ref, w2_ref, b2_ref,
              cid_ref, skip_ref, out_ref, z_s):
    i = pl.program_id(0)

    @pl.when(i == 0)
    def _():
        cnt = jnp.maximum(c0_ref[...] + c1_ref[...], 1.0)
        xp = sums_ref[...] / cnt
        a = jnp.minimum(pair_ref[0] + pair_ref[1], 1.0)
        degp = jnp.sum(a, axis=1, keepdims=True) + 1.0
        dinvp = lax.rsqrt(degp)
        xw = jnp.dot(xp, w2_ref[...], preferred_element_type=jnp.float32)
        t = dinvp * xw
        z_s[...] = dinvp * (jnp.dot(a, t, preferred_element_type=jnp.float32)
                            + t) + b2_ref[...]

    cid = cid_ref[0, 0, :]
    oh = (cid[:, None] == lax.broadcasted_iota(jnp.int32, (BM, K), 1)
          ).astype(jnp.float32)
    up = jnp.dot(oh, z_s[...], preferred_element_type=jnp.float32)
    out_ref[...] = up + skip_ref[...]


_out = pl.pallas_call(
    _out_body,
    grid=(GRID,),
    in_specs=[
        pl.BlockSpec((K, HID), lambda i: (0, 0)),
        pl.BlockSpec((K, 1), lambda i: (0, 0)),
        pl.BlockSpec((K, 1), lambda i: (0, 0)),
        pl.BlockSpec((2, K, K), lambda i: (0, 0, 0)),
        pl.BlockSpec((HID, OUT), lambda i: (0, 0)),
        pl.BlockSpec((1, OUT), lambda i: (0, 0)),
        pl.BlockSpec((1, 1, BM), lambda i: (i, 0, 0)),
        pl.BlockSpec((BM, OUT), lambda i: (i, 0)),
    ],
    out_specs=pl.BlockSpec((BM, OUT), lambda i: (i, 0)),
    out_shape=jax.ShapeDtypeStruct((N, OUT), jnp.float32),
    scratch_shapes=[pltpu.VMEM((K, OUT), jnp.float32)],
)


def kernel(x, edge_index, cluster_id, W1, b1, W2, b2, Ws, bs):
    src = jnp.concatenate(
        [edge_index[0], jnp.zeros((EPAD - E,), jnp.int32)])
    dst = jnp.concatenate(
        [edge_index[1], jnp.full((EPAD - E,), N, jnp.int32)])
    clu_pad = jnp.concatenate(
        [cluster_id, jnp.full((NPAD - N,), K, jnp.int32)])
    z1 = jnp.zeros((PZC,), jnp.float32)
    deg2, pair2, cnt2 = _stats(src, dst, clu_pad, z1)

    x2, dinv = _prep(x,
                     deg2[:N].reshape(N, 1),
                     deg2[NPAD:NPAD + N].reshape(N, 1))

    z2 = jnp.zeros((ROWS_PT, 128), jnp.float32)
    agg_flat = _agg(src, dst, x2.reshape(NC * NPAD, 128), z2)
    agg2 = agg_flat.reshape(NC, NPAD, 128)

    cid3 = cluster_id.reshape(GRID, 1, BM)
    skip, sums = _main(x, agg2, dinv, W1, b1.reshape(1, HID),
                       Ws, bs.reshape(1, OUT), cid3)

    pair3 = pair2.reshape(NC, PAIRPAD // K, K)
    logits = _out(sums,
                  cnt2[:K].reshape(K, 1),
                  cnt2[KC:KC + K].reshape(K, 1),
                  pair3,
                  W2, b2.reshape(1, OUT),
                  cid3, skip)
    return (logits, 0.0)
